# baseline (device time: 53398 ns/iter reference)
import jax
import jax.numpy as jnp
from jax import lax
from jax.experimental import pallas as pl
from jax.experimental.pallas import tpu as pltpu

N_DEV = 8

MASK_X, MASK_Y, MASK_Z = 1, 3, 4

GROUP_MASKS = (
    (MASK_X, MASK_Y, MASK_Z),
    (MASK_Y, MASK_Z, MASK_X),
    (MASK_Z, MASK_X, MASK_Y),
)
GROUP_ROWS = ((0, 384), (384, 384), (768, 256))
N_G = 3


def kernel(A, B):
    m, k = A.shape
    _, n = B.shape

    def body(a_ref, b_ref, out_ref, *scratch):
        rs_bufs = scratch[: 3 * N_G]
        send_sems = scratch[3 * N_G]
        recv_sems = scratch[3 * N_G + 1]

        my = lax.axis_index("i")
        vx = (my ^ (my >> 1)) & 1
        vy = (my >> 1) & 1
        vz = (my >> 2) & 1
        bit_of = {MASK_X: vx, MASK_Y: vy, MASK_Z: vz}

        barrier_sem = pltpu.get_barrier_semaphore()
        for mask in (MASK_X, MASK_Y, MASK_Z):
            pl.semaphore_signal(
                barrier_sem, inc=1,
                device_id=(my ^ mask,),
                device_id_type=pl.DeviceIdType.MESH,
            )
        pl.semaphore_wait(barrier_sem, 3)

        all_rdmas = []

        def rs_exchange_start(g, r, lo):
            base, S = GROUP_ROWS[g]
            half = S >> (r + 1)
            mask = GROUP_MASKS[g][r]
            bit = bit_of[mask]
            partner = my ^ mask
            send_lo = lo + (1 - bit) * half
            keep_lo = lo + bit * half
            rdma = pltpu.make_async_remote_copy(
                src_ref=out_ref.at[pl.ds(base + send_lo, half), :],
                dst_ref=rs_bufs[g * 3 + r],
                send_sem=send_sems.at[g * 3 + r],
                recv_sem=recv_sems.at[g * 3 + r],
                device_id=(partner,),
                device_id_type=pl.DeviceIdType.MESH,
            )
            rdma.start()
            all_rdmas.append(rdma)
            return rdma, keep_lo

        def ag_exchange_start(g, a, lo, ln):
            base, _ = GROUP_ROWS[g]
            mask = GROUP_MASKS[g][2 - a]
            partner = my ^ mask
            rdma = pltpu.make_async_remote_copy(
                src_ref=out_ref.at[pl.ds(base + lo, ln), :],
                dst_ref=out_ref.at[pl.ds(base + lo, ln), :],
                send_sem=send_sems.at[9 + g * 3 + a],
                recv_sem=recv_sems.at[9 + g * 3 + a],
                device_id=(partner,),
                device_id_type=pl.DeviceIdType.MESH,
            )
            rdma.start()
            all_rdmas.append(rdma)
            return rdma

        rdmas = [None] * N_G
        los = [None] * N_G
        for g in range(N_G):
            base, S = GROUP_ROWS[g]
            out_ref[base : base + S, :] = jnp.dot(
                a_ref[base : base + S, :],
                b_ref[:, :],
                preferred_element_type=jnp.float32,
            )
            rdmas[g], los[g] = rs_exchange_start(g, 0, 0)

        for r in range(3):
            for g in range(N_G):
                base, S = GROUP_ROWS[g]
                half = S >> (r + 1)
                rdmas[g].wait_recv()
                if r < 2:
                    out_ref[pl.ds(base + los[g], half), :] += rs_bufs[g * 3 + r][:, :]
                    rdmas[g], los[g] = rs_exchange_start(g, r + 1, los[g])
                else:
                    out_ref[pl.ds(base + los[g], half), :] = jnp.maximum(
                        out_ref[pl.ds(base + los[g], half), :]
                        + rs_bufs[g * 3 + r][:, :],
                        0.0,
                    )
                    rdmas[g] = ag_exchange_start(g, 0, los[g], S >> 3)

        for a in range(3):
            for g in range(N_G):
                _, S = GROUP_ROWS[g]
                ln = S >> (3 - a)
                bit = bit_of[GROUP_MASKS[g][2 - a]]
                rdmas[g].wait_recv()
                los[g] = los[g] - bit * ln
                if a < 2:
                    rdmas[g] = ag_exchange_start(g, a + 1, los[g], S >> (2 - a))

        for rdma in all_rdmas:
            rdma.wait_send()

    scratch_shapes = [
        pltpu.VMEM((S >> (r + 1), n), jnp.float32)
        for _, S in GROUP_ROWS
        for r in range(3)
    ] + [
        pltpu.SemaphoreType.DMA((18,)),
        pltpu.SemaphoreType.DMA((18,)),
    ]

    return pl.pallas_call(
        body,
        out_shape=jax.ShapeDtypeStruct((m, n), jnp.float32),
        in_specs=[
            pl.BlockSpec(memory_space=pltpu.VMEM),
            pl.BlockSpec(memory_space=pltpu.VMEM),
        ],
        out_specs=pl.BlockSpec(memory_space=pltpu.VMEM),
        scratch_shapes=scratch_shapes,
        compiler_params=pltpu.CompilerParams(
            vmem_limit_bytes=100 * 1024 * 1024,
            collective_id=0,
        ),
    )(A, B)


# device time: 45230 ns/iter; 1.1806x vs baseline; 1.1806x over previous
import jax
import jax.numpy as jnp
from jax import lax
from jax.experimental import pallas as pl
from jax.experimental.pallas import tpu as pltpu

N_DEV = 8

MASK_X, MASK_Y, MASK_Z = 1, 3, 4

ORDER_XYZ = (MASK_X, MASK_Y, MASK_Z)
ORDER_YZX = (MASK_Y, MASK_Z, MASK_X)
ORDER_ZXY = (MASK_Z, MASK_X, MASK_Y)

CHUNKS = (
    (0, 192, ORDER_XYZ),
    (384, 192, ORDER_YZX),
    (704, 192, ORDER_ZXY),
    (192, 192, ORDER_XYZ),
    (576, 128, ORDER_YZX),
    (896, 128, ORDER_ZXY),
)
N_C = len(CHUNKS)


def kernel(A, B):
    m, k = A.shape
    _, n = B.shape

    def body(a_ref, b_ref, out_ref, *scratch):
        rs_bufs = scratch[: 3 * N_C]
        send_sems = scratch[3 * N_C]
        recv_sems = scratch[3 * N_C + 1]

        my = lax.axis_index("i")
        vx = (my ^ (my >> 1)) & 1
        vy = (my >> 1) & 1
        vz = (my >> 2) & 1
        bit_of = {MASK_X: vx, MASK_Y: vy, MASK_Z: vz}

        barrier_sem = pltpu.get_barrier_semaphore()
        for mask in (MASK_X, MASK_Y, MASK_Z):
            pl.semaphore_signal(
                barrier_sem, inc=1,
                device_id=(my ^ mask,),
                device_id_type=pl.DeviceIdType.MESH,
            )
        pl.semaphore_wait(barrier_sem, 3)

        all_rdmas = []

        def rs_exchange_start(c, r, lo):
            base, S, masks = CHUNKS[c]
            half = S >> (r + 1)
            bit = bit_of[masks[r]]
            partner = my ^ masks[r]
            send_lo = lo + (1 - bit) * half
            keep_lo = lo + bit * half
            rdma = pltpu.make_async_remote_copy(
                src_ref=out_ref.at[pl.ds(base + send_lo, half), :],
                dst_ref=rs_bufs[c * 3 + r],
                send_sem=send_sems.at[c * 3 + r],
                recv_sem=recv_sems.at[c * 3 + r],
                device_id=(partner,),
                device_id_type=pl.DeviceIdType.MESH,
            )
            rdma.start()
            all_rdmas.append(rdma)
            return rdma, keep_lo

        def ag_exchange_start(c, a, lo, ln):
            base, _, masks = CHUNKS[c]
            partner = my ^ masks[2 - a]
            rdma = pltpu.make_async_remote_copy(
                src_ref=out_ref.at[pl.ds(base + lo, ln), :],
                dst_ref=out_ref.at[pl.ds(base + lo, ln), :],
                send_sem=send_sems.at[18 + c * 3 + a],
                recv_sem=recv_sems.at[18 + c * 3 + a],
                device_id=(partner,),
                device_id_type=pl.DeviceIdType.MESH,
            )
            rdma.start()
            all_rdmas.append(rdma)
            return rdma

        rdmas = [None] * N_C
        los = [None] * N_C
        for c in range(N_C):
            base, S, _ = CHUNKS[c]
            out_ref[base : base + S, :] = jnp.dot(
                a_ref[base : base + S, :],
                b_ref[:, :],
                preferred_element_type=jnp.float32,
            )
            rdmas[c], los[c] = rs_exchange_start(c, 0, 0)

        for r in range(3):
            for c in range(N_C):
                base, S, _ = CHUNKS[c]
                half = S >> (r + 1)
                rdmas[c].wait_recv()
                if r < 2:
                    out_ref[pl.ds(base + los[c], half), :] += rs_bufs[c * 3 + r][:, :]
                    rdmas[c], los[c] = rs_exchange_start(c, r + 1, los[c])
                else:
                    out_ref[pl.ds(base + los[c], half), :] = jnp.maximum(
                        out_ref[pl.ds(base + los[c], half), :]
                        + rs_bufs[c * 3 + r][:, :],
                        0.0,
                    )
                    rdmas[c] = ag_exchange_start(c, 0, los[c], S >> 3)

        for a in range(3):
            for c in range(N_C):
                _, S, masks = CHUNKS[c]
                ln = S >> (3 - a)
                bit = bit_of[masks[2 - a]]
                rdmas[c].wait_recv()
                los[c] = los[c] - bit * ln
                if a < 2:
                    rdmas[c] = ag_exchange_start(c, a + 1, los[c], S >> (2 - a))

        for rdma in all_rdmas:
            rdma.wait_send()

    scratch_shapes = [
        pltpu.VMEM((S >> (r + 1), n), jnp.float32)
        for _, S, _ in CHUNKS
        for r in range(3)
    ] + [
        pltpu.SemaphoreType.DMA((6 * N_C,)),
        pltpu.SemaphoreType.DMA((6 * N_C,)),
    ]

    return pl.pallas_call(
        body,
        out_shape=jax.ShapeDtypeStruct((m, n), jnp.float32),
        in_specs=[
            pl.BlockSpec(memory_space=pltpu.VMEM),
            pl.BlockSpec(memory_space=pltpu.VMEM),
        ],
        out_specs=pl.BlockSpec(memory_space=pltpu.VMEM),
        scratch_shapes=scratch_shapes,
        compiler_params=pltpu.CompilerParams(
            vmem_limit_bytes=100 * 1024 * 1024,
            collective_id=0,
        ),
    )(A, B)


# device time: 33096 ns/iter; 1.6134x vs baseline; 1.3666x over previous
import jax
import jax.numpy as jnp
from jax import lax
from jax.experimental import pallas as pl
from jax.experimental.pallas import tpu as pltpu

N_DEV = 8

MASK_X, MASK_Y, MASK_Z = 1, 3, 4

ORDER_XYZ = (MASK_X, MASK_Y, MASK_Z)
ORDER_YZX = (MASK_Y, MASK_Z, MASK_X)
ORDER_ZXY = (MASK_Z, MASK_X, MASK_Y)

CHUNK = 128
CHUNKS = (
    (0, ORDER_XYZ),
    (128, ORDER_YZX),
    (256, ORDER_ZXY),
    (384, ORDER_XYZ),
    (512, ORDER_YZX),
    (640, ORDER_ZXY),
    (768, ORDER_XYZ),
    (896, ORDER_YZX),
)
N_C = len(CHUNKS)


def kernel(A, B):
    m, k = A.shape
    _, n = B.shape

    def body(a_ref, b_ref, out_ref, work_ref, *scratch):
        rs_bufs = scratch[: 3 * N_C]
        send_sems = scratch[3 * N_C]
        recv_sems = scratch[3 * N_C + 1]

        my = lax.axis_index("i")
        vx = (my ^ (my >> 1)) & 1
        vy = (my >> 1) & 1
        vz = (my >> 2) & 1
        bit_of = {MASK_X: vx, MASK_Y: vy, MASK_Z: vz}

        barrier_sem = pltpu.get_barrier_semaphore()
        for mask in (MASK_X, MASK_Y, MASK_Z):
            pl.semaphore_signal(
                barrier_sem, inc=1,
                device_id=(my ^ mask,),
                device_id_type=pl.DeviceIdType.MESH,
            )
        pl.semaphore_wait(barrier_sem, 3)

        all_rdmas = []

        def rs_exchange_start(c, r, lo):
            base, masks = CHUNKS[c]
            half = CHUNK >> (r + 1)
            bit = bit_of[masks[r]]
            partner = my ^ masks[r]
            send_lo = lo + (1 - bit) * half
            keep_lo = lo + bit * half
            rdma = pltpu.make_async_remote_copy(
                src_ref=work_ref.at[pl.ds(base + send_lo, half), :],
                dst_ref=rs_bufs[c * 3 + r],
                send_sem=send_sems.at[c * 3 + r],
                recv_sem=recv_sems.at[c * 3 + r],
                device_id=(partner,),
                device_id_type=pl.DeviceIdType.MESH,
            )
            rdma.start()
            all_rdmas.append(rdma)
            return rdma, keep_lo

        def ag_exchange_start(c, a, lo, ln):
            base, masks = CHUNKS[c]
            partner = my ^ masks[2 - a]
            rdma = pltpu.make_async_remote_copy(
                src_ref=work_ref.at[pl.ds(base + lo, ln), :],
                dst_ref=work_ref.at[pl.ds(base + lo, ln), :],
                send_sem=send_sems.at[3 * N_C + c * 3 + a],
                recv_sem=recv_sems.at[3 * N_C + c * 3 + a],
                device_id=(partner,),
                device_id_type=pl.DeviceIdType.MESH,
            )
            rdma.start()
            all_rdmas.append(rdma)
            return rdma

        rdmas = [None] * N_C
        los = [None] * N_C
        for c in range(N_C):
            base, _ = CHUNKS[c]
            work_ref[base : base + CHUNK, :] = jnp.dot(
                a_ref[base : base + CHUNK, :],
                b_ref[:, :],
                preferred_element_type=jnp.float32,
            ).astype(jnp.bfloat16)
            rdmas[c], los[c] = rs_exchange_start(c, 0, 0)

        for r in range(3):
            for c in range(N_C):
                base, _ = CHUNKS[c]
                half = CHUNK >> (r + 1)
                rdmas[c].wait_recv()
                if r < 2:
                    work_ref[pl.ds(base + los[c], half), :] += rs_bufs[c * 3 + r][:, :]
                    rdmas[c], los[c] = rs_exchange_start(c, r + 1, los[c])
                else:
                    blk = jnp.maximum(
                        work_ref[pl.ds(base + los[c], half), :]
                        + rs_bufs[c * 3 + r][:, :],
                        jnp.bfloat16(0.0),
                    )
                    work_ref[pl.ds(base + los[c], half), :] = blk
                    rdmas[c] = ag_exchange_start(c, 0, los[c], CHUNK >> 3)
                    out_ref[pl.ds(base + los[c], half), :] = blk.astype(
                        jnp.float32
                    )

        for a in range(3):
            for c in range(N_C):
                base, masks = CHUNKS[c]
                ln = CHUNK >> (3 - a)
                bit = bit_of[masks[2 - a]]
                rdmas[c].wait_recv()
                p_lo = los[c] + (1 - 2 * bit) * ln
                los[c] = los[c] - bit * ln
                if a < 2:
                    rdmas[c] = ag_exchange_start(c, a + 1, los[c], CHUNK >> (2 - a))
                out_ref[pl.ds(base + p_lo, ln), :] = work_ref[
                    pl.ds(base + p_lo, ln), :
                ].astype(jnp.float32)

        for rdma in all_rdmas:
            rdma.wait_send()

    scratch_shapes = [
        pltpu.VMEM((CHUNK >> (r + 1), n), jnp.bfloat16)
        for _ in range(N_C)
        for r in range(3)
    ] + [
        pltpu.SemaphoreType.DMA((6 * N_C,)),
        pltpu.SemaphoreType.DMA((6 * N_C,)),
    ]

    return pl.pallas_call(
        body,
        out_shape=jax.ShapeDtypeStruct((m, n), jnp.float32),
        in_specs=[
            pl.BlockSpec(memory_space=pltpu.VMEM),
            pl.BlockSpec(memory_space=pltpu.VMEM),
        ],
        out_specs=pl.BlockSpec(memory_space=pltpu.VMEM),
        scratch_shapes=[pltpu.VMEM((m, n), jnp.bfloat16)] + scratch_shapes,
        compiler_params=pltpu.CompilerParams(
            vmem_limit_bytes=100 * 1024 * 1024,
            collective_id=0,
        ),
    )(A, B)


# device time: 31755 ns/iter; 1.6816x vs baseline; 1.0422x over previous
import jax
import jax.numpy as jnp
from jax import lax
from jax.experimental import pallas as pl
from jax.experimental.pallas import tpu as pltpu

N_DEV = 8

MASK_X, MASK_Y, MASK_Z = 1, 3, 4

ORDER_XYZ = (MASK_X, MASK_Y, MASK_Z)
ORDER_YZX = (MASK_Y, MASK_Z, MASK_X)
ORDER_ZXY = (MASK_Z, MASK_X, MASK_Y)

CHUNK = 128
CHUNKS = (
    (0, ORDER_XYZ),
    (128, ORDER_YZX),
    (256, ORDER_ZXY),
    (384, ORDER_XYZ),
    (512, ORDER_YZX),
    (640, ORDER_ZXY),
    (768, ORDER_XYZ),
    (896, ORDER_YZX),
)
N_C = len(CHUNKS)


def kernel(A, B):
    m, k = A.shape
    _, n = B.shape

    def body(a_ref, b_ref, out_hbm, work_ref, fout_ref, *scratch):
        rs_bufs = scratch[: 3 * N_C]
        send_sems = scratch[3 * N_C]
        recv_sems = scratch[3 * N_C + 1]
        copy_sems = scratch[3 * N_C + 2]

        my = lax.axis_index("i")
        vx = (my ^ (my >> 1)) & 1
        vy = (my >> 1) & 1
        vz = (my >> 2) & 1
        bit_of = {MASK_X: vx, MASK_Y: vy, MASK_Z: vz}

        barrier_sem = pltpu.get_barrier_semaphore()
        for mask in (MASK_X, MASK_Y, MASK_Z):
            pl.semaphore_signal(
                barrier_sem, inc=1,
                device_id=(my ^ mask,),
                device_id_type=pl.DeviceIdType.MESH,
            )

        all_rdmas = []
        copy_ops = []

        def rs_exchange_start(c, r, lo):
            base, masks = CHUNKS[c]
            half = CHUNK >> (r + 1)
            bit = bit_of[masks[r]]
            partner = my ^ masks[r]
            send_lo = lo + (1 - bit) * half
            keep_lo = lo + bit * half
            rdma = pltpu.make_async_remote_copy(
                src_ref=work_ref.at[pl.ds(base + send_lo, half), :],
                dst_ref=rs_bufs[c * 3 + r],
                send_sem=send_sems.at[c * 3 + r],
                recv_sem=recv_sems.at[c * 3 + r],
                device_id=(partner,),
                device_id_type=pl.DeviceIdType.MESH,
            )
            rdma.start()
            all_rdmas.append(rdma)
            return rdma, keep_lo

        def ag_exchange_start(c, a, lo, ln):
            base, masks = CHUNKS[c]
            partner = my ^ masks[2 - a]
            rdma = pltpu.make_async_remote_copy(
                src_ref=work_ref.at[pl.ds(base + lo, ln), :],
                dst_ref=work_ref.at[pl.ds(base + lo, ln), :],
                send_sem=send_sems.at[3 * N_C + c * 3 + a],
                recv_sem=recv_sems.at[3 * N_C + c * 3 + a],
                device_id=(partner,),
                device_id_type=pl.DeviceIdType.MESH,
            )
            rdma.start()
            all_rdmas.append(rdma)
            return rdma

        def emit_block(c, slot, lo, ln, values=None):
            base, _ = CHUNKS[c]
            if values is None:
                values = work_ref[pl.ds(base + lo, ln), :]
            fout_ref[pl.ds(base + lo, ln), :] = values.astype(jnp.float32)
            cp = pltpu.make_async_copy(
                fout_ref.at[pl.ds(base + lo, ln), :],
                out_hbm.at[pl.ds(base + lo, ln), :],
                copy_sems.at[c * 4 + slot],
            )
            cp.start()
            copy_ops.append(cp)

        rdmas = [None] * N_C
        los = [None] * N_C
        for c in range(N_C):
            base, _ = CHUNKS[c]
            work_ref[base : base + CHUNK, :] = jnp.dot(
                a_ref[base : base + CHUNK, :],
                b_ref[:, :],
                preferred_element_type=jnp.float32,
            ).astype(jnp.bfloat16)
            if c == 0:
                pl.semaphore_wait(barrier_sem, 3)
            rdmas[c], los[c] = rs_exchange_start(c, 0, 0)

        for r in range(3):
            for c in range(N_C):
                base, _ = CHUNKS[c]
                half = CHUNK >> (r + 1)
                rdmas[c].wait_recv()
                if r < 2:
                    work_ref[pl.ds(base + los[c], half), :] += rs_bufs[c * 3 + r][:, :]
                    rdmas[c], los[c] = rs_exchange_start(c, r + 1, los[c])
                else:
                    blk = jnp.maximum(
                        work_ref[pl.ds(base + los[c], half), :]
                        + rs_bufs[c * 3 + r][:, :],
                        jnp.bfloat16(0.0),
                    )
                    work_ref[pl.ds(base + los[c], half), :] = blk
                    rdmas[c] = ag_exchange_start(c, 0, los[c], CHUNK >> 3)
                    emit_block(c, 0, los[c], half, values=blk)

        for a in range(3):
            for c in range(N_C):
                base, masks = CHUNKS[c]
                ln = CHUNK >> (3 - a)
                bit = bit_of[masks[2 - a]]
                rdmas[c].wait_recv()
                p_lo = los[c] + (1 - 2 * bit) * ln
                los[c] = los[c] - bit * ln
                if a < 2:
                    rdmas[c] = ag_exchange_start(c, a + 1, los[c], CHUNK >> (2 - a))
                emit_block(c, a + 1, p_lo, ln)

        for rdma in all_rdmas:
            rdma.wait_send()
        for cp in copy_ops:
            cp.wait()

    scratch_shapes = [
        pltpu.VMEM((m, n), jnp.bfloat16),
        pltpu.VMEM((m, n), jnp.float32),
    ] + [
        pltpu.VMEM((CHUNK >> (r + 1), n), jnp.bfloat16)
        for _ in range(N_C)
        for r in range(3)
    ] + [
        pltpu.SemaphoreType.DMA((6 * N_C,)),
        pltpu.SemaphoreType.DMA((6 * N_C,)),
        pltpu.SemaphoreType.DMA((4 * N_C,)),
    ]

    return pl.pallas_call(
        body,
        out_shape=jax.ShapeDtypeStruct((m, n), jnp.float32),
        in_specs=[
            pl.BlockSpec(memory_space=pltpu.VMEM),
            pl.BlockSpec(memory_space=pltpu.VMEM),
        ],
        out_specs=pl.BlockSpec(memory_space=pl.ANY),
        scratch_shapes=scratch_shapes,
        compiler_params=pltpu.CompilerParams(
            vmem_limit_bytes=100 * 1024 * 1024,
            collective_id=0,
        ),
    )(A, B)


# device time: 31036 ns/iter; 1.7205x vs baseline; 1.0232x over previous
import jax
import jax.numpy as jnp
from jax import lax
from jax.experimental import pallas as pl
from jax.experimental.pallas import tpu as pltpu

N_DEV = 8

MASK_X, MASK_Y, MASK_Z = 1, 3, 4

ORDER_XYZ = (MASK_X, MASK_Y, MASK_Z)
ORDER_YZX = (MASK_Y, MASK_Z, MASK_X)
ORDER_ZXY = (MASK_Z, MASK_X, MASK_Y)

CHUNK = 128
SUB = CHUNK // 8
CHUNKS = (
    (0, ORDER_XYZ),
    (128, ORDER_YZX),
    (256, ORDER_ZXY),
    (384, ORDER_XYZ),
    (512, ORDER_YZX),
    (640, ORDER_ZXY),
    (768, ORDER_XYZ),
    (896, ORDER_YZX),
)
N_C = len(CHUNKS)

SLOTS = 8


def kernel(A, B):
    m, k = A.shape
    _, n = B.shape

    def body(a_ref, b_ref, out_hbm, work_ref, fout_ref, *scratch):
        r0_bufs = scratch[:N_C]
        r1_bufs = scratch[N_C : 4 * N_C]
        send_sems = scratch[4 * N_C]
        recv_sems = scratch[4 * N_C + 1]
        copy_sems = scratch[4 * N_C + 2]

        my = lax.axis_index("i")
        vx = (my ^ (my >> 1)) & 1
        vy = (my >> 1) & 1
        vz = (my >> 2) & 1
        bit_of = {MASK_X: vx, MASK_Y: vy, MASK_Z: vz}

        peers = (MASK_X, MASK_Y, MASK_Z, MASK_X ^ MASK_Y, MASK_Y ^ MASK_Z,
                 MASK_Z ^ MASK_X)
        barrier_sem = pltpu.get_barrier_semaphore()
        for mask in peers:
            pl.semaphore_signal(
                barrier_sem, inc=1,
                device_id=(my ^ mask,),
                device_id_type=pl.DeviceIdType.MESH,
            )

        all_rdmas = []
        copy_ops = []

        def start_rdma(src, dst, slot, partner):
            rdma = pltpu.make_async_remote_copy(
                src_ref=src,
                dst_ref=dst,
                send_sem=send_sems.at[slot],
                recv_sem=recv_sems.at[slot],
                device_id=(partner,),
                device_id_type=pl.DeviceIdType.MESH,
            )
            rdma.start()
            all_rdmas.append(rdma)
            return rdma

        def emit_block(c, slot, lo, ln, values=None):
            base, _ = CHUNKS[c]
            if values is None:
                values = work_ref[pl.ds(base + lo, ln), :]
            fout_ref[pl.ds(base + lo, ln), :] = values.astype(jnp.float32)
            cp = pltpu.make_async_copy(
                fout_ref.at[pl.ds(base + lo, ln), :],
                out_hbm.at[pl.ds(base + lo, ln), :],
                copy_sems.at[c * 5 + slot],
            )
            cp.start()
            copy_ops.append(cp)

        rdmas = [None] * N_C
        keep = [None] * N_C
        subs = [None] * N_C
        for c in range(N_C):
            base, masks = CHUNKS[c]
            work_ref[base : base + CHUNK, :] = jnp.dot(
                a_ref[base : base + CHUNK, :],
                b_ref[:, :],
                preferred_element_type=jnp.float32,
            ).astype(jnp.bfloat16)
            if c == 0:
                pl.semaphore_wait(barrier_sem, len(peers))
            b1 = bit_of[masks[0]]
            half = CHUNK // 2
            send_lo = (1 - b1) * half
            keep[c] = b1 * half
            rdmas[c] = start_rdma(
                work_ref.at[pl.ds(base + send_lo, half), :],
                r0_bufs[c],
                c * SLOTS + 0,
                my ^ masks[0],
            )

        for c in range(N_C):
            base, masks = CHUNKS[c]
            half = CHUNK // 2
            rdmas[c].wait_recv()
            work_ref[pl.ds(base + keep[c], half), :] += r0_bufs[c][:, :]
            b2, b3 = bit_of[masks[1]], bit_of[masks[2]]
            s_me = 2 * b2 + b3
            s2 = 2 * (1 - b2) + b3
            s3 = 2 * b2 + (1 - b3)
            s23 = 2 * (1 - b2) + (1 - b3)
            subs[c] = (s_me, s2, s3, s23)
            partners = (my ^ masks[1], my ^ masks[2], my ^ masks[1] ^ masks[2])
            group = []
            for j, (s, p) in enumerate(zip((s2, s3, s23), partners)):
                group.append(start_rdma(
                    work_ref.at[pl.ds(base + keep[c] + s * SUB, SUB), :],
                    r1_bufs[c * 3 + j],
                    c * SLOTS + 1 + j,
                    p,
                ))
            rdmas[c] = group

        for c in range(N_C):
            base, masks = CHUNKS[c]
            s_me, s2, s3, s23 = subs[c]
            for rdma in rdmas[c]:
                rdma.wait_recv()
            my_lo = keep[c] + s_me * SUB
            blk = jnp.maximum(
                work_ref[pl.ds(base + my_lo, SUB), :]
                + r1_bufs[c * 3 + 0][:, :]
                + r1_bufs[c * 3 + 1][:, :]
                + r1_bufs[c * 3 + 2][:, :],
                jnp.bfloat16(0.0),
            )
            work_ref[pl.ds(base + my_lo, SUB), :] = blk
            partners = (my ^ masks[1], my ^ masks[2], my ^ masks[1] ^ masks[2])
            group = []
            for j, p in enumerate(partners):
                group.append(start_rdma(
                    work_ref.at[pl.ds(base + my_lo, SUB), :],
                    work_ref.at[pl.ds(base + my_lo, SUB), :],
                    c * SLOTS + 4 + j,
                    p,
                ))
            rdmas[c] = group
            emit_block(c, 0, my_lo, SUB, values=blk)

        for c in range(N_C):
            base, masks = CHUNKS[c]
            s_me, s2, s3, s23 = subs[c]
            for rdma in rdmas[c]:
                rdma.wait_recv()
            half = CHUNK // 2
            rdmas[c] = start_rdma(
                work_ref.at[pl.ds(base + keep[c], half), :],
                work_ref.at[pl.ds(base + keep[c], half), :],
                c * SLOTS + 7,
                my ^ masks[0],
            )
            for j, s in enumerate((s2, s3, s23)):
                emit_block(c, 1 + j, keep[c] + s * SUB, SUB)

        for c in range(N_C):
            base, masks = CHUNKS[c]
            rdmas[c].wait_recv()
            b1 = bit_of[masks[0]]
            p_keep = (1 - b1) * (CHUNK // 2)
            emit_block(c, 4, p_keep, CHUNK // 2)

        for rdma in all_rdmas:
            rdma.wait_send()
        for cp in copy_ops:
            cp.wait()

    scratch_shapes = [
        pltpu.VMEM((m, n), jnp.bfloat16),
        pltpu.VMEM((m, n), jnp.float32),
    ] + [
        pltpu.VMEM((CHUNK // 2, n), jnp.bfloat16) for _ in range(N_C)
    ] + [
        pltpu.VMEM((SUB, n), jnp.bfloat16) for _ in range(3 * N_C)
    ] + [
        pltpu.SemaphoreType.DMA((SLOTS * N_C,)),
        pltpu.SemaphoreType.DMA((SLOTS * N_C,)),
        pltpu.SemaphoreType.DMA((5 * N_C,)),
    ]

    return pl.pallas_call(
        body,
        out_shape=jax.ShapeDtypeStruct((m, n), jnp.float32),
        in_specs=[
            pl.BlockSpec(memory_space=pltpu.VMEM),
            pl.BlockSpec(memory_space=pltpu.VMEM),
        ],
        out_specs=pl.BlockSpec(memory_space=pl.ANY),
        scratch_shapes=scratch_shapes,
        compiler_params=pltpu.CompilerParams(
            vmem_limit_bytes=100 * 1024 * 1024,
            collective_id=0,
        ),
    )(A, B)
